# baseline (speedup 1.0000x reference)
import functools

import jax
import jax.numpy as jnp
from jax import lax
from jax.experimental import pallas as pl
from jax.experimental.pallas import tpu as pltpu
from jax.experimental.pallas import tpu_sc as plsc

N_NODES = 10000
D_FEAT = 128
N_EDGES = 320000

NC = 2
NS = 16
NW = NC * NS
L = 16

PW = N_EDGES // NW
NBINS = 10240
RANGE = NBINS // NW
CH = 80
NFULL = PW // CH
TAIL = PW - NFULL * CH
NBUF = 8
GAHEAD = 4

_mesh = functools.partial(
    plsc.VectorSubcoreMesh, core_axis_name="c", subcore_axis_name="s")

_params = pltpu.CompilerParams(needs_layout_passes=False)


def _wid():
  return lax.axis_index("s") * NC + lax.axis_index("c")


def _iota():
  return lax.iota(jnp.int32, L)


def _hist_kernel(idxi_hbm, h_hbm, r_hbm, keys_v, hist_v, r_v):
  w = _wid()
  pltpu.sync_copy(idxi_hbm.at[pl.ds(w * PW, PW)], keys_v)

  @pl.loop(0, NBINS // L)
  def _(i):
    hist_v[pl.ds(pl.multiple_of(i * L, L), L)] = jnp.zeros((L,), jnp.int32)

  @pl.loop(0, PW // L, unroll=4)
  def _(i):
    k = keys_v[pl.ds(pl.multiple_of(i * L, L), L)]
    cnt, last = plsc.scan_count(k)
    plsc.addupdate_scatter(hist_v, [k], cnt, mask=last)

  iota = _iota()
  r0 = jnp.zeros((L,), jnp.int32)
  r1 = jnp.zeros((L,), jnp.int32)
  for u in range(NW):
    acc = jnp.zeros((L,), jnp.int32)
    for j in range(RANGE // L):
      acc = acc + hist_v[pl.ds(u * RANGE + j * L, L)]
    s = jnp.sum(acc)
    if u < L:
      r0 = jnp.where(iota == u, s, r0)
    else:
      r1 = jnp.where(iota == (u - L), s, r1)
  r_v[pl.ds(0, L)] = r0
  r_v[pl.ds(L, L)] = r1

  pltpu.sync_copy(hist_v, h_hbm.at[pl.ds(w * NBINS, NBINS)])
  pltpu.sync_copy(r_v, r_hbm.at[w])


def _offsets_kernel(h_hbm, r_hbm, b_hbm, rv, hc, bc, dsem):
  u = _wid()
  for w in range(NW):
    pltpu.async_copy(
        h_hbm.at[pl.ds(pl.multiple_of(w * NBINS + u * RANGE, L), RANGE)],
        hc.at[pl.ds(w * RANGE, RANGE)], dsem)
  pltpu.sync_copy(r_hbm, rv)
  pltpu.make_async_copy(h_hbm.at[pl.ds(0, NW * RANGE)], hc, dsem).wait()

  cs0 = jnp.zeros((L,), jnp.int32)
  cs1 = jnp.zeros((L,), jnp.int32)
  for w in range(NW):
    cs0 = cs0 + rv[w, pl.ds(0, L)]
    cs1 = cs1 + rv[w, pl.ds(L, L)]
  ic0 = plsc.cumsum(cs0)
  ic1 = plsc.cumsum(cs1) + jnp.sum(cs0)
  ex0 = ic0 - cs0
  ex1 = ic1 - cs1
  iota = _iota()
  g = (jnp.sum(jnp.where(iota == u, ex0, 0))
       + jnp.sum(jnp.where(iota == (u - L), ex1, 0)))

  for j in range(RANGE // L):
    t = jnp.zeros((L,), jnp.int32)
    for w in range(NW):
      t = t + hc[pl.ds(w * RANGE + j * L, L)]
    inc = plsc.cumsum(t)
    gv = (inc - t) + g
    t = jnp.zeros((L,), jnp.int32)
    for w in range(NW):
      bc[pl.ds(w * RANGE + j * L, L)] = gv + t
      t = t + hc[pl.ds(w * RANGE + j * L, L)]
    g = g + jnp.sum(t)

  for w in range(NW):
    pltpu.async_copy(
        bc.at[pl.ds(w * RANGE, RANGE)],
        b_hbm.at[pl.ds(pl.multiple_of(w * NBINS + u * RANGE, L), RANGE)],
        dsem)
  pltpu.make_async_copy(bc, b_hbm.at[pl.ds(0, NW * RANGE)], dsem).wait()


def _gather_kernel(x_hbm, idxi_hbm, idxj_hbm, b_hbm, out_hbm,
                   keys_v, idxj_v, brow_v, ranks_v, tailr_v,
                   rowbuf, tailbuf, gsem, ssem):
  w = _wid()
  base = w * PW

  pltpu.sync_copy(idxj_hbm.at[pl.ds(base, PW)], idxj_v)

  def gath(c):
    return pltpu.async_copy(
        x_hbm.at[idxj_v.at[pl.ds(c * CH, CH)]],
        rowbuf.at[lax.rem(c, NBUF)], gsem)

  def scat(c):
    return pltpu.async_copy(
        rowbuf.at[lax.rem(c, NBUF)], out_hbm.at[ranks_v.at[c]], ssem)

  for c in range(GAHEAD):
    gath(c)

  pltpu.sync_copy(idxi_hbm.at[pl.ds(base, PW)], keys_v)
  pltpu.sync_copy(b_hbm.at[pl.ds(w * NBINS, NBINS)], brow_v)

  def rank_chunk(c):
    for v in range(CH // L):
      k = keys_v[pl.ds(pl.multiple_of(c * CH + v * L, L), L)]
      cnt, last = plsc.scan_count(k)
      bse = plsc.load_gather(brow_v, [k])
      ranks_v[c, pl.ds(v * L, L)] = bse + cnt - 1
      plsc.addupdate_scatter(brow_v, [k], cnt, mask=last)

  def drain_s():
    pltpu.make_async_copy(rowbuf.at[0], out_hbm.at[pl.ds(0, CH)], ssem).wait()

  def drain_g():
    pltpu.make_async_copy(x_hbm.at[pl.ds(0, CH)], rowbuf.at[0], gsem).wait()

  SDEPTH = NBUF - GAHEAD
  for c in range(SDEPTH):
    rank_chunk(c)
    gath(c + GAHEAD)
    drain_g()
    scat(c)

  @pl.loop(SDEPTH, NFULL - GAHEAD)
  def _(c):
    rank_chunk(c)
    drain_s()
    gath(c + GAHEAD)
    drain_g()
    scat(c)

  for c in range(NFULL - GAHEAD, NFULL):
    rank_chunk(c)
    drain_s()
    drain_g()
    scat(c)

  if TAIL:
    k = keys_v[pl.ds(NFULL * CH, TAIL)]
    cnt, last = plsc.scan_count(k)
    bse = plsc.load_gather(brow_v, [k])
    tailr_v[...] = bse + cnt - 1

  for _ in range(SDEPTH - 1):
    drain_s()

  if TAIL:
    pltpu.async_copy(
        x_hbm.at[idxj_v.at[pl.ds(NFULL * CH, TAIL)]], tailbuf, gsem).wait()
    pltpu.async_copy(tailbuf, out_hbm.at[tailr_v], ssem).wait()
  drain_s()


def kernel(x, edge_index):
  ei = edge_index.astype(jnp.int32)
  idxi = ei[0]
  idxj = ei[1]

  hist = pl.kernel(
      _hist_kernel,
      out_type=(jax.ShapeDtypeStruct((NW * NBINS,), jnp.int32),
                jax.ShapeDtypeStruct((NW, NW), jnp.int32)),
      mesh=_mesh(),
      compiler_params=_params,
      scratch_types=[
          pltpu.VMEM((PW,), jnp.int32),
          pltpu.VMEM((NBINS,), jnp.int32),
          pltpu.VMEM((NW,), jnp.int32),
      ])
  h, r = hist(idxi)

  offs = pl.kernel(
      _offsets_kernel,
      out_type=jax.ShapeDtypeStruct((NW * NBINS,), jnp.int32),
      mesh=_mesh(),
      compiler_params=_params,
      scratch_types=[
          pltpu.VMEM((NW, NW), jnp.int32),
          pltpu.VMEM((NW * RANGE,), jnp.int32),
          pltpu.VMEM((NW * RANGE,), jnp.int32),
          pltpu.SemaphoreType.DMA,
      ])
  b = offs(h, r)

  gather = pl.kernel(
      _gather_kernel,
      out_type=jax.ShapeDtypeStruct((N_EDGES, D_FEAT), jnp.float32),
      mesh=_mesh(),
      compiler_params=_params,
      scratch_types=[
          pltpu.VMEM((PW,), jnp.int32),
          pltpu.VMEM((PW,), jnp.int32),
          pltpu.VMEM((NBINS,), jnp.int32),
          pltpu.VMEM((NFULL, CH), jnp.int32),
          pltpu.VMEM((max(TAIL, L),), jnp.int32),
          pltpu.VMEM((NBUF, CH, D_FEAT), jnp.float32),
          pltpu.VMEM((max(TAIL, L), D_FEAT), jnp.float32),
          pltpu.SemaphoreType.DMA,
          pltpu.SemaphoreType.DMA,
      ])
  return gather(x, idxi, idxj, b)


# speedup vs baseline: 1.3730x; 1.3730x over previous
import functools

import jax
import jax.numpy as jnp
from jax import lax
from jax.experimental import pallas as pl
from jax.experimental.pallas import tpu as pltpu
from jax.experimental.pallas import tpu_sc as plsc

N_NODES = 10000
D_FEAT = 128
N_EDGES = 320000

NC = 2
NS = 16
NW = NC * NS
L = 16

PW = N_EDGES // NW
NBINS = 10240
RANGE = NBINS // NW
CH = 80
NFULL = PW // CH
NBUF = 3
GAHEAD = 2
BLK = 2000
CPB = BLK // CH
NBLK = PW // BLK
SR = 6

_mesh = functools.partial(
    plsc.VectorSubcoreMesh, core_axis_name="c", subcore_axis_name="s")

_params = pltpu.CompilerParams(needs_layout_passes=False)


def _wid():
  return lax.axis_index("s") * NC + lax.axis_index("c")


def _iota():
  return lax.iota(jnp.int32, L)


def _hist_kernel(idxi_hbm, h_hbm, r_hbm, keys_v, hist_v, r_v):
  w = _wid()
  pltpu.sync_copy(idxi_hbm.at[pl.ds(w * PW, PW)], keys_v)

  @pl.loop(0, NBINS // L)
  def _(i):
    hist_v[pl.ds(pl.multiple_of(i * L, L), L)] = jnp.zeros((L,), jnp.int32)

  @pl.loop(0, PW // L, unroll=4)
  def _(i):
    k = keys_v[pl.ds(pl.multiple_of(i * L, L), L)]
    cnt, last = plsc.scan_count(k)
    plsc.addupdate_scatter(hist_v, [k], cnt, mask=last)

  iota = _iota()
  r0 = jnp.zeros((L,), jnp.int32)
  r1 = jnp.zeros((L,), jnp.int32)
  for u in range(NW):
    acc = jnp.zeros((L,), jnp.int32)
    for j in range(RANGE // L):
      acc = acc + hist_v[pl.ds(u * RANGE + j * L, L)]
    s = jnp.sum(acc)
    if u < L:
      r0 = jnp.where(iota == u, s, r0)
    else:
      r1 = jnp.where(iota == (u - L), s, r1)
  r_v[pl.ds(0, L)] = r0
  r_v[pl.ds(L, L)] = r1

  pltpu.sync_copy(hist_v, h_hbm.at[pl.ds(w * NBINS, NBINS)])
  pltpu.sync_copy(r_v, r_hbm.at[w])


def _offsets_kernel(h_hbm, r_hbm, b_hbm, rv, hc, bc, dsem):
  u = _wid()
  for w in range(NW):
    pltpu.async_copy(
        h_hbm.at[pl.ds(pl.multiple_of(w * NBINS + u * RANGE, L), RANGE)],
        hc.at[pl.ds(w * RANGE, RANGE)], dsem)
  pltpu.sync_copy(r_hbm, rv)
  pltpu.make_async_copy(h_hbm.at[pl.ds(0, NW * RANGE)], hc, dsem).wait()

  cs0 = jnp.zeros((L,), jnp.int32)
  cs1 = jnp.zeros((L,), jnp.int32)
  for w in range(NW):
    cs0 = cs0 + rv[w, pl.ds(0, L)]
    cs1 = cs1 + rv[w, pl.ds(L, L)]
  ic0 = plsc.cumsum(cs0)
  ic1 = plsc.cumsum(cs1) + jnp.sum(cs0)
  ex0 = ic0 - cs0
  ex1 = ic1 - cs1
  iota = _iota()
  g = (jnp.sum(jnp.where(iota == u, ex0, 0))
       + jnp.sum(jnp.where(iota == (u - L), ex1, 0)))

  for j in range(RANGE // L):
    t = jnp.zeros((L,), jnp.int32)
    for w in range(NW):
      t = t + hc[pl.ds(w * RANGE + j * L, L)]
    inc = plsc.cumsum(t)
    gv = (inc - t) + g
    t = jnp.zeros((L,), jnp.int32)
    for w in range(NW):
      bc[pl.ds(w * RANGE + j * L, L)] = gv + t
      t = t + hc[pl.ds(w * RANGE + j * L, L)]
    g = g + jnp.sum(t)

  for w in range(NW):
    pltpu.async_copy(
        bc.at[pl.ds(w * RANGE, RANGE)],
        b_hbm.at[pl.ds(pl.multiple_of(w * NBINS + u * RANGE, L), RANGE)],
        dsem)
  pltpu.make_async_copy(bc, b_hbm.at[pl.ds(0, NW * RANGE)], dsem).wait()


def _gather_kernel(x_hbm, idxi_hbm, idxj_hbm, b_hbm, out_hbm,
                   keys_blk, idxj_blk, brow_v, sranks, rowbuf,
                   x_spm, gsem, ssem, bsem, xsem):
  w = _wid()
  sid = lax.axis_index("s")
  base = w * PW

  @pl.when(sid == 0)
  def _():
    pltpu.async_copy(x_hbm, x_spm, xsem)

  def load_blk(b):
    off = lax.rem(b, 2) * BLK
    pltpu.async_copy(idxi_hbm.at[pl.ds(base + b * BLK, BLK)],
                     keys_blk.at[pl.ds(off, BLK)], bsem)
    pltpu.async_copy(idxj_hbm.at[pl.ds(base + b * BLK, BLK)],
                     idxj_blk.at[pl.ds(off, BLK)], bsem)

  def wait_blk():
    for _ in range(2):
      pltpu.make_async_copy(idxi_hbm.at[pl.ds(0, BLK)],
                            keys_blk.at[pl.ds(0, BLK)], bsem).wait()

  load_blk(0)
  pltpu.sync_copy(b_hbm.at[pl.ds(w * NBINS, NBINS)], brow_v)
  wait_blk()
  load_blk(1)

  def rank_chunk(c):
    off = lax.rem(lax.div(c, CPB), 2) * BLK + lax.rem(c, CPB) * CH
    srow = lax.rem(c, SR)
    for v in range(CH // L):
      k = keys_blk[pl.ds(off + v * L, L)]
      cnt, last = plsc.scan_count(k)
      bse = plsc.load_gather(brow_v, [k])
      sranks[srow, pl.ds(v * L, L)] = bse + cnt - 1
      plsc.addupdate_scatter(brow_v, [k], cnt, mask=last)

  def gath(c):
    off = lax.rem(lax.div(c, CPB), 2) * BLK + lax.rem(c, CPB) * CH
    return pltpu.async_copy(
        x_spm.at[idxj_blk.at[pl.ds(off, CH)]],
        rowbuf.at[lax.rem(c, NBUF)], gsem)

  def scat(c):
    return pltpu.async_copy(
        rowbuf.at[lax.rem(c, NBUF)], out_hbm.at[sranks.at[lax.rem(c, SR)]],
        ssem)

  def drain_s():
    pltpu.make_async_copy(rowbuf.at[0], out_hbm.at[pl.ds(0, CH)], ssem).wait()

  def drain_g():
    pltpu.make_async_copy(x_hbm.at[pl.ds(0, CH)], rowbuf.at[0], gsem).wait()

  for c in range(GAHEAD):
    rank_chunk(c)

  @pl.when(sid == 0)
  def _():
    pltpu.make_async_copy(x_hbm, x_spm, xsem).wait()
  plsc.subcore_barrier()

  for c in range(GAHEAD):
    gath(c)

  SDEPTH = NBUF - GAHEAD

  @pl.loop(0, NFULL)
  def _(c):
    b = lax.div(c, CPB)
    cin = lax.rem(c, CPB)

    @pl.when((cin == CPB - GAHEAD) & (b + 1 < NBLK))
    def _():
      wait_blk()

    cr = c + GAHEAD
    @pl.when(cr < NFULL)
    def _():
      rank_chunk(cr)

    @pl.when(c >= SDEPTH)
    def _():
      drain_s()

    @pl.when(c + GAHEAD < NFULL)
    def _():
      gath(c + GAHEAD)

    drain_g()
    scat(c)

    @pl.when((cin == SDEPTH) & (b >= 1) & (b + 1 < NBLK))
    def _():
      load_blk(b + 1)

  for _ in range(SDEPTH):
    drain_s()


def kernel(x, edge_index):
  ei = edge_index.astype(jnp.int32)
  idxi = ei[0]
  idxj = ei[1]

  hist = pl.kernel(
      _hist_kernel,
      out_type=(jax.ShapeDtypeStruct((NW * NBINS,), jnp.int32),
                jax.ShapeDtypeStruct((NW, NW), jnp.int32)),
      mesh=_mesh(),
      compiler_params=_params,
      scratch_types=[
          pltpu.VMEM((PW,), jnp.int32),
          pltpu.VMEM((NBINS,), jnp.int32),
          pltpu.VMEM((NW,), jnp.int32),
      ])
  h, r = hist(idxi)

  offs = pl.kernel(
      _offsets_kernel,
      out_type=jax.ShapeDtypeStruct((NW * NBINS,), jnp.int32),
      mesh=_mesh(),
      compiler_params=_params,
      scratch_types=[
          pltpu.VMEM((NW, NW), jnp.int32),
          pltpu.VMEM((NW * RANGE,), jnp.int32),
          pltpu.VMEM((NW * RANGE,), jnp.int32),
          pltpu.SemaphoreType.DMA,
      ])
  b = offs(h, r)

  gather = pl.kernel(
      _gather_kernel,
      out_type=jax.ShapeDtypeStruct((N_EDGES, D_FEAT), jnp.float32),
      mesh=_mesh(),
      compiler_params=_params,
      scratch_types=[
          pltpu.VMEM((2 * BLK,), jnp.int32),
          pltpu.VMEM((2 * BLK,), jnp.int32),
          pltpu.VMEM((NBINS,), jnp.int32),
          pltpu.VMEM((SR, CH), jnp.int32),
          pltpu.VMEM((NBUF, CH, D_FEAT), jnp.float32),
          pltpu.VMEM_SHARED((N_NODES, D_FEAT), jnp.float32),
          pltpu.SemaphoreType.DMA,
          pltpu.SemaphoreType.DMA,
          pltpu.SemaphoreType.DMA,
          pltpu.SemaphoreType.DMA,
      ])
  return gather(x, idxi, idxj, b)
